# baseline (speedup 1.0000x reference)
import functools

import jax
import jax.numpy as jnp
from jax import lax
from jax.experimental import pallas as pl
from jax.experimental.pallas import tpu as pltpu
from jax.experimental.pallas import tpu_sc as plsc

NE = 1000000
D = 32
B, S = 16384, 50
NC, NS = 2, 16
NW = NC * NS
FULL_COLS = NE // 128
TAIL = NE - FULL_COLS * 128
K1_UNITS = FULL_COLS // NW + 1
NBLK = (S * (B // 128)) // NW

_TC_TILED = pltpu.CompilerParams(
    use_tc_tiling_on_sc=True, needs_layout_passes=False
)


def _mesh():
    return plsc.VectorSubcoreMesh(core_axis_name="c", subcore_axis_name="s")


def _pack_unit(in_v, tr_v, nrows, iota_lo, iota_hi):
    def body(p, _):
        for q in range(4):
            col = jnp.full((16,), 0, jnp.int32) + (4 * p + q)
            lo = plsc.load_gather(in_v, [iota_lo, col])
            hi = plsc.load_gather(in_v, [iota_hi, col])
            tr_v[p, pl.ds(32 * q, 16)] = lo
            tr_v[p, pl.ds(32 * q + 16, 16)] = hi
        return ()

    lax.fori_loop(0, nrows, body, ())


def _k1(emb_t, tail_packed):
    @functools.partial(
        pl.kernel,
        mesh=_mesh(),
        out_type=jax.ShapeDtypeStruct((NE // 4, 128), jnp.float32),
        scratch_types=[
            pltpu.VMEM((D, 128), jnp.float32),
            pltpu.VMEM((D, 128), jnp.float32),
            pltpu.VMEM((TAIL // 4, 128), jnp.float32),
        ],
        compiler_params=_TC_TILED,
    )
    def k1(emb_hbm, tail_hbm, scr_hbm, in_v, tr_v, tl_v):
        wid = lax.axis_index("s") * NC + lax.axis_index("c")
        iota_lo = jax.lax.iota(jnp.int32, 16)
        iota_hi = iota_lo + 16

        @pl.when(wid == NW - 1)
        def _():
            pltpu.sync_copy(tail_hbm, tl_v)
            pltpu.sync_copy(tl_v, scr_hbm.at[pl.ds(FULL_COLS * 32, TAIL // 4), :])

        def unit(u, _):
            j = wid + NW * u

            @pl.when(j < FULL_COLS)
            def _():
                pltpu.sync_copy(emb_hbm.at[:, pl.ds(j * 128, 128)], in_v)
                _pack_unit(in_v, tr_v, 32, iota_lo, iota_hi)
                pltpu.sync_copy(tr_v, scr_hbm.at[pl.ds(j * 32, 32), :])

            return ()

        lax.fori_loop(0, K1_UNITS, unit, ())

    return k1(emb_t, tail_packed)


def _k2(scratch, tok_t):
    @functools.partial(
        pl.kernel,
        mesh=_mesh(),
        out_type=jax.ShapeDtypeStruct((S, D, B), jnp.float32),
        scratch_types=[
            pltpu.VMEM((2, 128, 128), jnp.float32),
            pltpu.VMEM((2, D, 128), jnp.float32),
            pltpu.VMEM((2, 128), jnp.int32),
            pltpu.VMEM((2, 128), jnp.int32),
            pltpu.VMEM((2, 128), jnp.int32),
            pltpu.SemaphoreType.DMA,
            pltpu.SemaphoreType.DMA,
            pltpu.SemaphoreType.DMA,
        ],
        compiler_params=_TC_TILED,
    )
    def k2(scr_hbm, tok_hbm, out_hbm, g_v, st_v, tk_v, ix_v, of_v,
           gsem, tsem, wsem):
        wid = lax.axis_index("s") * NC + lax.axis_index("c")
        iota_lo = jax.lax.iota(jnp.int32, 16)

        def blk(m):
            n = wid + NW * m
            return n // 128, lax.rem(n, 128)

        def prefetch_tok(m, half):
            s, jb = blk(m)
            pltpu.async_copy(
                tok_hbm.at[s, pl.ds(jb * 128, 128)], tk_v.at[half], tsem
            )

        def wait_tok(half):
            pltpu.make_async_copy(
                tok_hbm.at[0, pl.ds(0, 128)], tk_v.at[half], tsem
            ).wait()

        def split_ids(half):
            for q in range(8):
                t = tk_v[half, pl.ds(16 * q, 16)]
                ix_v[half, pl.ds(16 * q, 16)] = ((t >> 13) << 11) + (t & 2047)
                of_v[half, pl.ds(16 * q, 16)] = ((t >> 11) & 3) * 32

        def issue_gathers(half):
            for q in range(4):
                pltpu.async_copy(
                    scr_hbm.at[ix_v.at[half, pl.ds(q * 32, 32)]],
                    g_v.at[half, pl.ds(q * 32, 32)],
                    gsem,
                )

        def drain_gathers(half):
            for q in range(4):
                pltpu.make_async_copy(
                    scr_hbm.at[ix_v.at[half, pl.ds(q * 32, 32)]],
                    g_v.at[half, pl.ds(q * 32, 32)],
                    gsem,
                ).wait()

        def wait_write():
            pltpu.make_async_copy(
                st_v.at[0], out_hbm.at[0, :, pl.ds(0, 128)], wsem
            ).wait()

        prefetch_tok(0, 0)
        wait_tok(0)
        split_ids(0)
        issue_gathers(0)
        prefetch_tok(1, 1)

        def pair(p, _):
            for i in (0, 1):
                b = 2 * p + i
                half, other = i, 1 - i
                drain_gathers(half)
                wait_tok(other)
                split_ids(other)
                issue_gathers(other)
                prefetch_tok(lax.rem(b + 2, NBLK), half)

                for q in range(8):
                    rows = iota_lo + 16 * q
                    off = of_v[half, pl.ds(16 * q, 16)]
                    for d in range(D):
                        vals = plsc.load_gather(
                            g_v.at[half], [rows, off + d]
                        )
                        st_v[half, d, pl.ds(16 * q, 16)] = vals

                if i == 0:
                    @pl.when(b > 0)
                    def _():
                        wait_write()
                else:
                    wait_write()
                s, jb = blk(b)
                pltpu.async_copy(
                    st_v.at[half], out_hbm.at[s, :, pl.ds(jb * 128, 128)], wsem
                )
            return ()

        lax.fori_loop(0, NBLK // 2, pair, ())

        drain_gathers(0)
        wait_tok(0)
        wait_write()

    return k2(scratch, tok_t)


NSTRIPE = (NE + 8191) // 8192
NROWS = NSTRIPE * 2048


def _repack_tc(emb_t):
    def body(x_ref, o_ref):
        x = x_ref[...]
        o_ref[...] = jnp.concatenate(
            [x[:, 2048 * q:2048 * (q + 1)] for q in range(4)], axis=0
        ).T

    return pl.pallas_call(
        body,
        grid=(NSTRIPE,),
        in_specs=[pl.BlockSpec((D, 8192), lambda j: (0, j))],
        out_specs=pl.BlockSpec((2048, 128), lambda j: (j, 0)),
        out_shape=jax.ShapeDtypeStruct((NROWS, 128), jnp.float32),
    )(emb_t)


def kernel(token_ids, embeddings):
    scratch = _repack_tc(embeddings.T)
    o_t = _k2(scratch, token_ids.T.astype(jnp.int32))
    return jnp.transpose(o_t, (2, 0, 1))


# speedup vs baseline: 1.0505x; 1.0505x over previous
import functools

import jax
import jax.numpy as jnp
from jax import lax
from jax.experimental import pallas as pl
from jax.experimental.pallas import tpu as pltpu
from jax.experimental.pallas import tpu_sc as plsc

NE = 1000000
D = 32
B, S = 16384, 50
NC, NS = 2, 16
NW = NC * NS
FULL_COLS = NE // 128
TAIL = NE - FULL_COLS * 128
K1_UNITS = FULL_COLS // NW + 1
NBLK = (S * (B // 128)) // NW

_TC_TILED = pltpu.CompilerParams(
    use_tc_tiling_on_sc=True, needs_layout_passes=False
)


def _mesh():
    return plsc.VectorSubcoreMesh(core_axis_name="c", subcore_axis_name="s")


def _pack_unit(in_v, tr_v, nrows, iota_lo, iota_hi):
    def body(p, _):
        for q in range(4):
            col = jnp.full((16,), 0, jnp.int32) + (4 * p + q)
            lo = plsc.load_gather(in_v, [iota_lo, col])
            hi = plsc.load_gather(in_v, [iota_hi, col])
            tr_v[p, pl.ds(32 * q, 16)] = lo
            tr_v[p, pl.ds(32 * q + 16, 16)] = hi
        return ()

    lax.fori_loop(0, nrows, body, ())


def _k1(emb_t, tail_packed):
    @functools.partial(
        pl.kernel,
        mesh=_mesh(),
        out_type=jax.ShapeDtypeStruct((NE // 4, 128), jnp.float32),
        scratch_types=[
            pltpu.VMEM((D, 128), jnp.float32),
            pltpu.VMEM((D, 128), jnp.float32),
            pltpu.VMEM((TAIL // 4, 128), jnp.float32),
        ],
        compiler_params=_TC_TILED,
    )
    def k1(emb_hbm, tail_hbm, scr_hbm, in_v, tr_v, tl_v):
        wid = lax.axis_index("s") * NC + lax.axis_index("c")
        iota_lo = jax.lax.iota(jnp.int32, 16)
        iota_hi = iota_lo + 16

        @pl.when(wid == NW - 1)
        def _():
            pltpu.sync_copy(tail_hbm, tl_v)
            pltpu.sync_copy(tl_v, scr_hbm.at[pl.ds(FULL_COLS * 32, TAIL // 4), :])

        def unit(u, _):
            j = wid + NW * u

            @pl.when(j < FULL_COLS)
            def _():
                pltpu.sync_copy(emb_hbm.at[:, pl.ds(j * 128, 128)], in_v)
                _pack_unit(in_v, tr_v, 32, iota_lo, iota_hi)
                pltpu.sync_copy(tr_v, scr_hbm.at[pl.ds(j * 32, 32), :])

            return ()

        lax.fori_loop(0, K1_UNITS, unit, ())

    return k1(emb_t, tail_packed)


def _k2(scratch, tok_t):
    @functools.partial(
        pl.kernel,
        mesh=_mesh(),
        out_type=jax.ShapeDtypeStruct((S, D, B), jnp.float32),
        scratch_types=[
            pltpu.VMEM((2, 128, 128), jnp.float32),
            pltpu.VMEM((2, D, 128), jnp.float32),
            pltpu.VMEM((2, 128), jnp.int32),
            pltpu.VMEM((2, 128), jnp.int32),
            pltpu.VMEM((2, 128), jnp.int32),
            pltpu.SemaphoreType.DMA,
            pltpu.SemaphoreType.DMA,
            pltpu.SemaphoreType.DMA,
        ],
        compiler_params=_TC_TILED,
    )
    def k2(scr_hbm, tok_hbm, out_hbm, g_v, st_v, tk_v, ix_v, of_v,
           gsem, tsem, wsem):
        wid = lax.axis_index("s") * NC + lax.axis_index("c")
        iota_lo = jax.lax.iota(jnp.int32, 16)

        def blk(m):
            n = wid + NW * m
            return n // 128, lax.rem(n, 128)

        def prefetch_tok(m, half):
            s, jb = blk(m)
            pltpu.async_copy(
                tok_hbm.at[s, pl.ds(jb * 128, 128)], tk_v.at[half], tsem
            )

        def wait_tok(half):
            pltpu.make_async_copy(
                tok_hbm.at[0, pl.ds(0, 128)], tk_v.at[half], tsem
            ).wait()

        def split_ids(half):
            for q in range(8):
                t = tk_v[half, pl.ds(16 * q, 16)]
                ix_v[half, pl.ds(16 * q, 16)] = ((t >> 13) << 11) + (t & 2047)
                of_v[half, pl.ds(16 * q, 16)] = ((t >> 11) & 3) * 32

        def issue_gathers(half):
            pltpu.async_copy(
                scr_hbm.at[ix_v.at[half, pl.ds(0, 128)]],
                g_v.at[half],
                gsem,
            )

        def drain_gathers(half):
            pltpu.make_async_copy(
                scr_hbm.at[ix_v.at[half, pl.ds(0, 128)]],
                g_v.at[half],
                gsem,
            ).wait()

        def wait_write():
            pltpu.make_async_copy(
                st_v.at[0], out_hbm.at[0, :, pl.ds(0, 128)], wsem
            ).wait()

        prefetch_tok(0, 0)
        wait_tok(0)
        split_ids(0)
        issue_gathers(0)
        prefetch_tok(1, 1)

        def pair(p, _):
            for i in (0, 1):
                b = 2 * p + i
                half, other = i, 1 - i
                drain_gathers(half)
                wait_tok(other)
                split_ids(other)
                issue_gathers(other)
                prefetch_tok(lax.rem(b + 2, NBLK), half)

                def tr(q, _):
                    rows = iota_lo + 16 * q
                    off = of_v[half, pl.ds(16 * q, 16)]
                    for d in range(D):
                        vals = plsc.load_gather(
                            g_v.at[half], [rows, off + d]
                        )
                        st_v[half, d, pl.ds(16 * q, 16)] = vals
                    return ()

                lax.fori_loop(0, 8, tr, ())

                if i == 0:
                    @pl.when(b > 0)
                    def _():
                        wait_write()
                else:
                    wait_write()
                s, jb = blk(b)
                pltpu.async_copy(
                    st_v.at[half], out_hbm.at[s, :, pl.ds(jb * 128, 128)], wsem
                )
            return ()

        lax.fori_loop(0, NBLK // 2, pair, ())

        drain_gathers(0)
        wait_tok(0)
        wait_write()

    return k2(scratch, tok_t)


NSTRIPE = (NE + 8191) // 8192
NROWS = NSTRIPE * 2048


def _repack_tc(emb_t):
    def body(x_ref, o_ref):
        x = x_ref[...]
        o_ref[...] = jnp.concatenate(
            [x[:, 2048 * q:2048 * (q + 1)] for q in range(4)], axis=0
        ).T

    return pl.pallas_call(
        body,
        grid=(NSTRIPE,),
        in_specs=[pl.BlockSpec((D, 8192), lambda j: (0, j))],
        out_specs=pl.BlockSpec((2048, 128), lambda j: (j, 0)),
        out_shape=jax.ShapeDtypeStruct((NROWS, 128), jnp.float32),
    )(emb_t)


def kernel(token_ids, embeddings):
    scratch = _repack_tc(embeddings.T)
    o_t = _k2(scratch, token_ids.T.astype(jnp.int32))
    return jnp.transpose(o_t, (2, 0, 1))


# speedup vs baseline: 1.0509x; 1.0004x over previous
import functools

import jax
import jax.numpy as jnp
from jax import lax
from jax.experimental import pallas as pl
from jax.experimental.pallas import tpu as pltpu
from jax.experimental.pallas import tpu_sc as plsc

NE = 1000000
D = 32
B, S = 16384, 50
NC, NS = 2, 16
NW = NC * NS
FULL_COLS = NE // 128
TAIL = NE - FULL_COLS * 128
K1_UNITS = FULL_COLS // NW + 1
TB = 256
NBLK = (S * (B // TB)) // NW

_TC_TILED = pltpu.CompilerParams(
    use_tc_tiling_on_sc=True, needs_layout_passes=False
)


def _mesh():
    return plsc.VectorSubcoreMesh(core_axis_name="c", subcore_axis_name="s")


def _pack_unit(in_v, tr_v, nrows, iota_lo, iota_hi):
    def body(p, _):
        for q in range(4):
            col = jnp.full((16,), 0, jnp.int32) + (4 * p + q)
            lo = plsc.load_gather(in_v, [iota_lo, col])
            hi = plsc.load_gather(in_v, [iota_hi, col])
            tr_v[p, pl.ds(32 * q, 16)] = lo
            tr_v[p, pl.ds(32 * q + 16, 16)] = hi
        return ()

    lax.fori_loop(0, nrows, body, ())


def _k1(emb_t, tail_packed):
    @functools.partial(
        pl.kernel,
        mesh=_mesh(),
        out_type=jax.ShapeDtypeStruct((NE // 4, 128), jnp.float32),
        scratch_types=[
            pltpu.VMEM((D, 128), jnp.float32),
            pltpu.VMEM((D, 128), jnp.float32),
            pltpu.VMEM((TAIL // 4, 128), jnp.float32),
        ],
        compiler_params=_TC_TILED,
    )
    def k1(emb_hbm, tail_hbm, scr_hbm, in_v, tr_v, tl_v):
        wid = lax.axis_index("s") * NC + lax.axis_index("c")
        iota_lo = jax.lax.iota(jnp.int32, 16)
        iota_hi = iota_lo + 16

        @pl.when(wid == NW - 1)
        def _():
            pltpu.sync_copy(tail_hbm, tl_v)
            pltpu.sync_copy(tl_v, scr_hbm.at[pl.ds(FULL_COLS * 32, TAIL // 4), :])

        def unit(u, _):
            j = wid + NW * u

            @pl.when(j < FULL_COLS)
            def _():
                pltpu.sync_copy(emb_hbm.at[:, pl.ds(j * 128, 128)], in_v)
                _pack_unit(in_v, tr_v, 32, iota_lo, iota_hi)
                pltpu.sync_copy(tr_v, scr_hbm.at[pl.ds(j * 32, 32), :])

            return ()

        lax.fori_loop(0, K1_UNITS, unit, ())

    return k1(emb_t, tail_packed)


def _k2(scratch, tok_t):
    @functools.partial(
        pl.kernel,
        mesh=_mesh(),
        out_type=jax.ShapeDtypeStruct((S, D, B), jnp.float32),
        scratch_types=[
            pltpu.VMEM((2, TB, 128), jnp.float32),
            pltpu.VMEM((2, D, TB), jnp.float32),
            pltpu.VMEM((2, TB), jnp.int32),
            pltpu.VMEM((2, TB), jnp.int32),
            pltpu.VMEM((2, TB), jnp.int32),
            pltpu.SemaphoreType.DMA,
            pltpu.SemaphoreType.DMA,
            pltpu.SemaphoreType.DMA,
        ],
        compiler_params=_TC_TILED,
    )
    def k2(scr_hbm, tok_hbm, out_hbm, g_v, st_v, tk_v, ix_v, of_v,
           gsem, tsem, wsem):
        wid = lax.axis_index("s") * NC + lax.axis_index("c")
        iota_lo = jax.lax.iota(jnp.int32, 16)

        def blk(m):
            n = wid + NW * m
            return n // (B // TB), lax.rem(n, B // TB)

        def prefetch_tok(m, half):
            s, jb = blk(m)
            pltpu.async_copy(
                tok_hbm.at[s, pl.ds(jb * TB, TB)], tk_v.at[half], tsem
            )

        def wait_tok(half):
            pltpu.make_async_copy(
                tok_hbm.at[0, pl.ds(0, TB)], tk_v.at[half], tsem
            ).wait()

        def split_ids(half):
            for q in range(TB // 16):
                t = tk_v[half, pl.ds(16 * q, 16)]
                ix_v[half, pl.ds(16 * q, 16)] = ((t >> 13) << 11) + (t & 2047)
                of_v[half, pl.ds(16 * q, 16)] = ((t >> 11) & 3) * 32

        def issue_gathers(half):
            for c in range(TB // 128):
                pltpu.async_copy(
                    scr_hbm.at[ix_v.at[half, pl.ds(128 * c, 128)]],
                    g_v.at[half, pl.ds(128 * c, 128)],
                    gsem,
                )

        def drain_gathers(half):
            for c in range(TB // 128):
                pltpu.make_async_copy(
                    scr_hbm.at[ix_v.at[half, pl.ds(128 * c, 128)]],
                    g_v.at[half, pl.ds(128 * c, 128)],
                    gsem,
                ).wait()

        def wait_write():
            pltpu.make_async_copy(
                st_v.at[0], out_hbm.at[0, :, pl.ds(0, TB)], wsem
            ).wait()

        prefetch_tok(0, 0)
        wait_tok(0)
        split_ids(0)
        issue_gathers(0)
        prefetch_tok(1, 1)

        def pair(p, _):
            for i in (0, 1):
                b = 2 * p + i
                half, other = i, 1 - i
                drain_gathers(half)
                wait_tok(other)
                split_ids(other)
                issue_gathers(other)
                prefetch_tok(lax.rem(b + 2, NBLK), half)

                def tr(q, _):
                    rows = iota_lo + 16 * q
                    off = of_v[half, pl.ds(16 * q, 16)]
                    for d in range(D):
                        vals = plsc.load_gather(
                            g_v.at[half], [rows, off + d]
                        )
                        st_v[half, d, pl.ds(16 * q, 16)] = vals
                    return ()

                lax.fori_loop(0, TB // 16, tr, ())

                if i == 0:
                    @pl.when(b > 0)
                    def _():
                        wait_write()
                else:
                    wait_write()
                s, jb = blk(b)
                pltpu.async_copy(
                    st_v.at[half], out_hbm.at[s, :, pl.ds(jb * TB, TB)], wsem
                )
            return ()

        lax.fori_loop(0, NBLK // 2, pair, ())

        drain_gathers(0)
        wait_tok(0)
        wait_write()

    return k2(scratch, tok_t)


NSTRIPE = (NE + 8191) // 8192
NROWS = NSTRIPE * 2048


def _repack_tc(emb_t):
    def body(x_ref, o_ref):
        x = x_ref[...]
        o_ref[...] = jnp.concatenate(
            [x[:, 2048 * q:2048 * (q + 1)] for q in range(4)], axis=0
        ).T

    return pl.pallas_call(
        body,
        grid=(NSTRIPE,),
        in_specs=[pl.BlockSpec((D, 8192), lambda j: (0, j))],
        out_specs=pl.BlockSpec((2048, 128), lambda j: (j, 0)),
        out_shape=jax.ShapeDtypeStruct((NROWS, 128), jnp.float32),
    )(emb_t)


def kernel(token_ids, embeddings):
    scratch = _repack_tc(embeddings.T)
    o_t = _k2(scratch, token_ids.T.astype(jnp.int32))
    return jnp.transpose(o_t, (2, 0, 1))


# speedup vs baseline: 1.1028x; 1.0494x over previous
import functools

import jax
import jax.numpy as jnp
from jax import lax
from jax.experimental import pallas as pl
from jax.experimental.pallas import tpu as pltpu
from jax.experimental.pallas import tpu_sc as plsc

NE = 1000000
D = 32
B, S = 16384, 50
NC, NS = 2, 16
NW = NC * NS
TB = 256
NBLK = (S * (B // TB)) // NW

_TC_TILED = pltpu.CompilerParams(
    use_tc_tiling_on_sc=True, needs_layout_passes=False
)


def _mesh():
    return plsc.VectorSubcoreMesh(core_axis_name="c", subcore_axis_name="s")


def _k2(scratch, tok_t):
    @functools.partial(
        pl.kernel,
        mesh=_mesh(),
        out_type=jax.ShapeDtypeStruct((S, D, B), jnp.float32),
        scratch_types=[
            pltpu.VMEM((2, TB, 128), jnp.float32),
            pltpu.VMEM((2, D, TB), jnp.float32),
            pltpu.VMEM((2, TB), jnp.int32),
            pltpu.VMEM((2, TB), jnp.int32),
            pltpu.VMEM((2, TB), jnp.int32),
            pltpu.SemaphoreType.DMA,
            pltpu.SemaphoreType.DMA,
            pltpu.SemaphoreType.DMA,
        ],
        compiler_params=_TC_TILED,
    )
    def k2(scr_hbm, tok_hbm, out_hbm, g_v, st_v, tk_v, ix_v, of_v,
           gsem, tsem, wsem):
        wid = lax.axis_index("s") * NC + lax.axis_index("c")
        iota_lo = jax.lax.iota(jnp.int32, 16)

        def blk(m):
            n = wid + NW * m
            return n // (B // TB), lax.rem(n, B // TB)

        def prefetch_tok(m, half):
            s, jb = blk(m)
            pltpu.async_copy(
                tok_hbm.at[s, pl.ds(jb * TB, TB)], tk_v.at[half], tsem
            )

        def wait_tok(half):
            pltpu.make_async_copy(
                tok_hbm.at[0, pl.ds(0, TB)], tk_v.at[half], tsem
            ).wait()

        def split_ids(half):
            for q in range(TB // 16):
                t = tk_v[half, pl.ds(16 * q, 16)]
                ix_v[half, pl.ds(16 * q, 16)] = ((t >> 14) << 12) + (t & (SQ - 1))
                of_v[half, pl.ds(16 * q, 16)] = ((t >> 12) & 3) * 32

        def issue_gathers(half):
            for c in range(TB // 128):
                pltpu.async_copy(
                    scr_hbm.at[ix_v.at[half, pl.ds(128 * c, 128)]],
                    g_v.at[half, pl.ds(128 * c, 128)],
                    gsem,
                )

        def drain_gathers(half):
            for c in range(TB // 128):
                pltpu.make_async_copy(
                    scr_hbm.at[ix_v.at[half, pl.ds(128 * c, 128)]],
                    g_v.at[half, pl.ds(128 * c, 128)],
                    gsem,
                ).wait()

        def wait_write():
            pltpu.make_async_copy(
                st_v.at[0], out_hbm.at[0, :, pl.ds(0, TB)], wsem
            ).wait()

        prefetch_tok(0, 0)
        wait_tok(0)
        split_ids(0)
        issue_gathers(0)
        prefetch_tok(1, 1)

        def pair(p, _):
            for i in (0, 1):
                b = 2 * p + i
                half, other = i, 1 - i
                drain_gathers(half)
                wait_tok(other)
                split_ids(other)
                issue_gathers(other)
                prefetch_tok(lax.rem(b + 2, NBLK), half)

                def tr(q, _):
                    rows = iota_lo + 16 * q
                    off = of_v[half, pl.ds(16 * q, 16)]
                    for d in range(D):
                        vals = plsc.load_gather(
                            g_v.at[half], [rows, off + d]
                        )
                        st_v[half, d, pl.ds(16 * q, 16)] = vals
                    return ()

                lax.fori_loop(0, TB // 16, tr, ())

                if i == 0:
                    @pl.when(b > 0)
                    def _():
                        wait_write()
                else:
                    wait_write()
                s, jb = blk(b)
                pltpu.async_copy(
                    st_v.at[half], out_hbm.at[s, :, pl.ds(jb * TB, TB)], wsem
                )
            return ()

        lax.fori_loop(0, NBLK // 2, pair, ())

        drain_gathers(0)
        wait_tok(0)
        wait_write()

    return k2(scratch, tok_t)


SW = 16384
SQ = SW // 4
NSTRIPE = (NE + SW - 1) // SW
NROWS = NSTRIPE * SQ


def _repack_tc(emb_t):
    def body(x_ref, o_ref):
        x = x_ref[...]
        o_ref[...] = jnp.concatenate(
            [x[:, SQ * q:SQ * (q + 1)] for q in range(4)], axis=0
        ).T

    return pl.pallas_call(
        body,
        grid=(NSTRIPE,),
        in_specs=[pl.BlockSpec((D, SW), lambda j: (0, j))],
        out_specs=pl.BlockSpec((SQ, 128), lambda j: (j, 0)),
        out_shape=jax.ShapeDtypeStruct((NROWS, 128), jnp.float32),
    )(emb_t)


def kernel(token_ids, embeddings):
    scratch = _repack_tc(embeddings.T)
    o_t = _k2(scratch, token_ids.T.astype(jnp.int32))
    return jnp.transpose(o_t, (2, 0, 1))
